# baseline (speedup 1.0000x reference)
import functools

import jax
import jax.numpy as jnp
import numpy as np
from jax import lax
from jax.experimental import pallas as pl
from jax.experimental.pallas import tpu as pltpu
from jax.experimental.pallas import tpu_sc as plsc

N = 10000
E = 320000
DF = 128
DE = 16
H = 8
HD = 16
SCALE = HD ** (-0.5)

NC = 2
NS = 16
NT = NC * NS
C = 32
G = C // 16
SUP = 12
NPAIR = SUP // 2
NSUP = 26
BCH = NSUP * SUP
NP_ = 10112
RS = NP_ // NS


def _proj_body(q_ref, k_ref, v_ref, wq_ref, wk_ref, wv_ref,
               qh_ref, kh_ref, vh_ref):
    qh_ref[...] = jnp.dot(q_ref[...], wq_ref[...],
                          preferred_element_type=jnp.float32) * SCALE
    kh_ref[...] = jnp.dot(k_ref[...], wk_ref[...],
                          preferred_element_type=jnp.float32)
    vh_ref[...] = jnp.dot(v_ref[...], wv_ref[...],
                          preferred_element_type=jnp.float32)


def _proj(q, k, v, Wq, Wk, Wv):
    BN = 2000
    grid = (N // BN,)
    bspec_x = pl.BlockSpec((BN, DF), lambda i: (i, 0))
    bspec_w = pl.BlockSpec((DF, DF), lambda i: (0, 0))
    return pl.pallas_call(
        _proj_body,
        grid=grid,
        in_specs=[bspec_x, bspec_x, bspec_x, bspec_w, bspec_w, bspec_w],
        out_specs=[bspec_x, bspec_x, bspec_x],
        out_shape=[jax.ShapeDtypeStruct((N, DF), jnp.float32)] * 3,
    )(q, k, v, Wq, Wk, Wv)


def _bias_body(e_ref, wb_ref, bb_ref, o_ref):
    o_ref[...] = jnp.dot(e_ref[...], wb_ref[...],
                         preferred_element_type=jnp.float32) + bb_ref[...]


def _bias(edges, Wb, bb):
    BE = 20000
    grid = (E // BE,)
    return pl.pallas_call(
        _bias_body,
        grid=grid,
        in_specs=[pl.BlockSpec((BE, DE), lambda i: (i, 0)),
                  pl.BlockSpec((DE, H), lambda i: (0, 0)),
                  pl.BlockSpec((1, H), lambda i: (0, 0))],
        out_specs=pl.BlockSpec((BE, H), lambda i: (i, 0)),
        out_shape=jax.ShapeDtypeStruct((E, H), jnp.float32),
    )(edges, Wb, bb.reshape(1, H))


def _sc_body(qh_hbm, kh_hbm, vh_hbm, edges_hbm, wbt_hbm, bbr_hbm,
             src2_hbm, dst2_hbm,
             acc_out, den_out,
             src2, dst2, edges2, wbtv, bbrv, qA, kA, vA, qB, kB, vB, exA, exB,
             acc_sh, den_sh,
             gq0, gk0, gv0, gq1, gk1, gv1, sac0, sde0, sac1, sde1):
    c = lax.axis_index("c")
    s = lax.axis_index("s")
    tile = c * NS + s
    base_chunk = tile * BCH + jnp.minimum(tile, 16)

    iota = lax.iota(jnp.int32, 16)
    zero16 = jnp.zeros((16,), jnp.float32)

    def _zero_vrow(r, _):
        for j in range(DF // 16):
            vA[r, pl.ds(j * 16, 16)] = zero16
        exA[r, pl.ds(0, 16)] = zero16
        exB[r, pl.ds(0, 16)] = zero16
        return 0

    lax.fori_loop(0, C, _zero_vrow, 0)
    pltpu.sync_copy(wbt_hbm, wbtv)
    pltpu.sync_copy(bbr_hbm, bbrv)

    row0 = s * RS
    for b in range(RS // C):
        pltpu.sync_copy(vA, acc_sh.at[pl.ds(row0 + b * C, C)])
        pltpu.sync_copy(exA, den_sh.at[pl.ds(row0 + b * C, C)])
    rtail = RS % C
    pltpu.sync_copy(vA.at[pl.ds(0, rtail)],
                    acc_sh.at[pl.ds(row0 + RS - rtail, rtail)])
    pltpu.sync_copy(exA.at[pl.ds(0, rtail)],
                    den_sh.at[pl.ds(row0 + RS - rtail, rtail)])
    plsc.subcore_barrier()

    def _issue(r, qb, kb, vb, sq, sk, sv):
        pltpu.async_copy(qh_hbm.at[src2.at[r]], qb, sq)
        pltpu.async_copy(kh_hbm.at[dst2.at[r]], kb, sk)
        pltpu.async_copy(vh_hbm.at[dst2.at[r]], vb, sv)

    def _wait_g(qb, kb, vb, sq, sk, sv):
        pltpu.make_async_copy(qh_hbm.at[src2.at[0]], qb, sq).wait()
        pltpu.make_async_copy(kh_hbm.at[dst2.at[0]], kb, sk).wait()
        pltpu.make_async_copy(vh_hbm.at[dst2.at[0]], vb, sv).wait()

    def _compute(r, qb, kb, vb, exbuf):
        eoff = r * (C * DE)

        @plsc.parallel_loop(0, G * H, unroll=2)
        def _dots(i):
            g = i >> 3
            h = i & 7
            rows = g * 16 + iota
            hb = h * HD
            ebase = eoff + rows * DE
            att0 = plsc.load_gather(bbrv, [hb + iota])
            att1 = jnp.zeros((16,), jnp.float32)
            for f in range(0, DE, 2):
                r0 = (f + iota) & (DE - 1)
                r1 = (f + 1 + iota) & (DE - 1)
                att0 = att0 + (plsc.load_gather(edges2, [ebase + r0])
                               * plsc.load_gather(wbtv, [hb + r0]))
                att1 = att1 + (plsc.load_gather(edges2, [ebase + r1])
                               * plsc.load_gather(wbtv, [hb + r1]))
            for d in range(0, HD, 2):
                c0 = ((d + iota) & (HD - 1)) + hb
                c1 = ((d + 1 + iota) & (HD - 1)) + hb
                att0 = att0 + (plsc.load_gather(qb, [rows, c0])
                               * plsc.load_gather(kb, [rows, c0]))
                att1 = att1 + (plsc.load_gather(qb, [rows, c1])
                               * plsc.load_gather(kb, [rows, c1]))
            ex = jnp.exp(att0 + att1)
            plsc.store_scatter(exbuf, [rows, jnp.full((16,), 0, jnp.int32) + h], ex)

        @plsc.parallel_loop(0, G * H, unroll=2)
        def _scale(i):
            g = i >> 3
            h = i & 7
            rows = g * 16 + iota
            hb = h * HD
            ex = plsc.load_gather(exbuf, [rows, jnp.full((16,), 0, jnp.int32) + h])
            for d in range(HD):
                col = ((d + iota) & (HD - 1)) + hb
                vc = plsc.load_gather(vb, [rows, col])
                plsc.store_scatter(vb, [rows, col], vc * ex)

    def _scat(r, vb, exbuf, sa, sd):
        pltpu.async_copy(vb, acc_sh.at[src2.at[r]], sa, add=True)
        pltpu.async_copy(exbuf, den_sh.at[src2.at[r]], sd, add=True)

    def _wait_scat(vb, exbuf, sa, sd):
        pltpu.make_async_copy(vb, acc_sh.at[src2.at[0]], sa).wait()
        pltpu.make_async_copy(exbuf, den_sh.at[src2.at[0]], sd).wait()

    def _sup(k, _):
        off = base_chunk + k * SUP
        pltpu.sync_copy(src2_hbm.at[pl.ds(off, SUP)], src2)
        pltpu.sync_copy(dst2_hbm.at[pl.ds(off, SUP)], dst2)
        pltpu.sync_copy(edges_hbm.at[pl.ds(off * (C * DE), SUP * C * DE)], edges2)
        _issue(0, qA, kA, vA, gq0, gk0, gv0)
        _issue(1, qB, kB, vB, gq1, gk1, gv1)

        def _pair(j, _):
            a = 2 * j
            _wait_g(qA, kA, vA, gq0, gk0, gv0)
            _compute(a, qA, kA, vA, exA)
            _scat(a, vA, exA, sac0, sde0)
            _wait_g(qB, kB, vB, gq1, gk1, gv1)
            _wait_scat(vA, exA, sac0, sde0)

            @pl.when(j < NPAIR - 1)
            def _():
                _issue(a + 2, qA, kA, vA, gq0, gk0, gv0)

            _compute(a + 1, qB, kB, vB, exB)
            _scat(a + 1, vB, exB, sac1, sde1)
            _wait_scat(vB, exB, sac1, sde1)

            @pl.when(j < NPAIR - 1)
            def _():
                _issue(a + 3, qB, kB, vB, gq1, gk1, gv1)

            return 0

        lax.fori_loop(0, NPAIR, _pair, 0)
        return 0

    lax.fori_loop(0, NSUP, _sup, 0)

    @pl.when(tile < 16)
    def _():
        off = base_chunk + BCH
        pltpu.sync_copy(src2_hbm.at[pl.ds(off, 1)], src2.at[pl.ds(0, 1)])
        pltpu.sync_copy(dst2_hbm.at[pl.ds(off, 1)], dst2.at[pl.ds(0, 1)])
        pltpu.sync_copy(edges_hbm.at[pl.ds(off * (C * DE), C * DE)],
                        edges2.at[pl.ds(0, C * DE)])
        _issue(0, qA, kA, vA, gq0, gk0, gv0)
        _wait_g(qA, kA, vA, gq0, gk0, gv0)
        _compute(0, qA, kA, vA, exA)
        _scat(0, vA, exA, sac0, sde0)
        _wait_scat(vA, exA, sac0, sde0)

    plsc.subcore_barrier()

    pltpu.sync_copy(acc_sh.at[pl.ds(row0, RS)], acc_out.at[c, pl.ds(row0, RS)])
    pltpu.sync_copy(den_sh.at[pl.ds(row0, RS)], den_out.at[c, pl.ds(row0, RS)])


def _sc_pass(qh, kh, vh, edges1d, wbt, bbr, src, dst):
    mesh = plsc.VectorSubcoreMesh(core_axis_name="c", subcore_axis_name="s")
    f = pl.kernel(
        _sc_body,
        out_type=(jax.ShapeDtypeStruct((NC, NP_, DF), jnp.float32),
                  jax.ShapeDtypeStruct((NC, NP_, 2 * H), jnp.float32)),
        mesh=mesh,
        compiler_params=pltpu.CompilerParams(needs_layout_passes=False,
                                             use_tc_tiling_on_sc=False),
        scratch_types=[
            pltpu.VMEM((SUP, C), jnp.int32),
            pltpu.VMEM((SUP, C), jnp.int32),
            pltpu.VMEM((SUP * C * DE,), jnp.float32),
            pltpu.VMEM((H * HD,), jnp.float32),
            pltpu.VMEM((H * HD,), jnp.float32),
            pltpu.VMEM((C, DF), jnp.float32),
            pltpu.VMEM((C, DF), jnp.float32),
            pltpu.VMEM((C, DF), jnp.float32),
            pltpu.VMEM((C, DF), jnp.float32),
            pltpu.VMEM((C, DF), jnp.float32),
            pltpu.VMEM((C, DF), jnp.float32),
            pltpu.VMEM((C, 2 * H), jnp.float32),
            pltpu.VMEM((C, 2 * H), jnp.float32),
            pltpu.VMEM_SHARED((NP_, DF), jnp.float32),
            pltpu.VMEM_SHARED((NP_, 2 * H), jnp.float32),
        ] + [pltpu.SemaphoreType.DMA] * 10,
    )
    return f(qh, kh, vh, edges1d, wbt, bbr,
             src.reshape(E // C, C), dst.reshape(E // C, C))


def _final_body(acc_ref, den_ref, r_ref, wo_ref, bo_ref, o_ref):
    a = acc_ref[0] + acc_ref[1]
    dn = den_ref[0] + den_ref[1]
    dr = jnp.dot(dn, r_ref[...], preferred_element_type=jnp.float32)
    dr = jnp.where(dr == 0.0, 1.0, dr)
    o = a / dr
    o_ref[...] = jnp.dot(o, wo_ref[...],
                         preferred_element_type=jnp.float32) + bo_ref[...]


def _finalize(acc, den, Wo, bo):
    BN = 2000
    grid = (N // BN,)
    rep = jnp.asarray(
        np.vstack([np.kron(np.eye(H), np.ones((1, HD))),
                   np.zeros((H, DF))]), dtype=jnp.float32)
    return pl.pallas_call(
        _final_body,
        grid=grid,
        in_specs=[pl.BlockSpec((NC, BN, DF), lambda i: (0, i, 0)),
                  pl.BlockSpec((NC, BN, 2 * H), lambda i: (0, i, 0)),
                  pl.BlockSpec((2 * H, DF), lambda i: (0, 0)),
                  pl.BlockSpec((DF, DF), lambda i: (0, 0)),
                  pl.BlockSpec((1, DF), lambda i: (0, 0))],
        out_specs=pl.BlockSpec((BN, DF), lambda i: (i, 0)),
        out_shape=jax.ShapeDtypeStruct((N, DF), jnp.float32),
    )(acc, den, rep, Wo, bo.reshape(1, DF))


def kernel(q, k, v, edges, edge_index, Wq, Wk, Wv, Wo, bo, Wb, bb):
    src = edge_index[:, 0]
    dst = edge_index[:, 1]
    qh, kh, vh = _proj(q, k, v, Wq, Wk, Wv)
    wbt = Wb.T.reshape(H * HD)
    bbr = jnp.repeat(bb, HD)
    acc, den = _sc_pass(qh, kh, vh, edges.reshape(E * DE), wbt, bbr, src, dst)
    return _finalize(acc, den, Wo, bo)


# speedup vs baseline: 1.2109x; 1.2109x over previous
import functools

import jax
import jax.numpy as jnp
import numpy as np
from jax import lax
from jax.experimental import pallas as pl
from jax.experimental.pallas import tpu as pltpu
from jax.experimental.pallas import tpu_sc as plsc

N = 10000
E = 320000
DF = 128
DE = 16
H = 8
HD = 16
SCALE = HD ** (-0.5)

NC = 2
NS = 16
NT = NC * NS
C = 32
G = C // 16
SUP = 12
NPAIR = SUP // 2
NSUP = 26
BCH = NSUP * SUP
NP_ = 10112
RS = NP_ // NS


def _proj_body(q_ref, k_ref, v_ref, wq_ref, wk_ref, wv_ref,
               qh_ref, kh_ref, vh_ref):
    qh_ref[...] = jnp.dot(q_ref[...], wq_ref[...],
                          preferred_element_type=jnp.float32) * SCALE
    kh_ref[...] = jnp.dot(k_ref[...], wk_ref[...],
                          preferred_element_type=jnp.float32)
    vh_ref[...] = jnp.dot(v_ref[...], wv_ref[...],
                          preferred_element_type=jnp.float32)


def _proj(q, k, v, Wq, Wk, Wv):
    BN = 2000
    grid = (N // BN,)
    bspec_x = pl.BlockSpec((BN, DF), lambda i: (i, 0))
    bspec_w = pl.BlockSpec((DF, DF), lambda i: (0, 0))
    return pl.pallas_call(
        _proj_body,
        grid=grid,
        in_specs=[bspec_x, bspec_x, bspec_x, bspec_w, bspec_w, bspec_w],
        out_specs=[bspec_x, bspec_x, bspec_x],
        out_shape=[jax.ShapeDtypeStruct((N, DF), jnp.float32)] * 3,
    )(q, k, v, Wq, Wk, Wv)


def _bias_body(e_ref, wb_ref, bb_ref, o_ref):
    o_ref[...] = jnp.dot(e_ref[...], wb_ref[...],
                         preferred_element_type=jnp.float32) + bb_ref[...]


def _bias(edges, Wb, bb):
    BE = 20000
    grid = (E // BE,)
    return pl.pallas_call(
        _bias_body,
        grid=grid,
        in_specs=[pl.BlockSpec((BE, DE), lambda i: (i, 0)),
                  pl.BlockSpec((DE, H), lambda i: (0, 0)),
                  pl.BlockSpec((1, H), lambda i: (0, 0))],
        out_specs=pl.BlockSpec((BE, H), lambda i: (i, 0)),
        out_shape=jax.ShapeDtypeStruct((E, H), jnp.float32),
    )(edges, Wb, bb.reshape(1, H))


def _sc_body(qh_hbm, kh_hbm, vh_hbm, edgesT_hbm, wbt_hbm, bbr_hbm,
             src2_hbm, dst2_hbm,
             acc_out, den_out,
             src2, dst2, edges2, wbtv, bbrv, qA, kA, vA, qB, kB, vB, exA, exB,
             acc_sh, den_sh,
             gq0, gk0, gv0, gq1, gk1, gv1, sac0, sde0, sac1, sde1):
    c = lax.axis_index("c")
    s = lax.axis_index("s")
    tile = c * NS + s
    base_chunk = tile * BCH + jnp.minimum(tile, 16)

    iota = lax.iota(jnp.int32, 16)
    zero16 = jnp.zeros((16,), jnp.float32)

    def _zero_vrow(r, _):
        for j in range(DF // 16):
            vA[r, pl.ds(j * 16, 16)] = zero16
        exA[r, pl.ds(0, 16)] = zero16
        exB[r, pl.ds(0, 16)] = zero16
        return 0

    lax.fori_loop(0, C, _zero_vrow, 0)
    pltpu.sync_copy(wbt_hbm, wbtv)
    pltpu.sync_copy(bbr_hbm, bbrv)

    row0 = s * RS
    for b in range(RS // C):
        pltpu.sync_copy(vA, acc_sh.at[pl.ds(row0 + b * C, C)])
        pltpu.sync_copy(exA, den_sh.at[pl.ds(row0 + b * C, C)])
    rtail = RS % C
    pltpu.sync_copy(vA.at[pl.ds(0, rtail)],
                    acc_sh.at[pl.ds(row0 + RS - rtail, rtail)])
    pltpu.sync_copy(exA.at[pl.ds(0, rtail)],
                    den_sh.at[pl.ds(row0 + RS - rtail, rtail)])
    plsc.subcore_barrier()

    def _issue(r, qb, kb, vb, sq, sk, sv):
        pltpu.async_copy(qh_hbm.at[src2.at[r]], qb, sq)
        pltpu.async_copy(kh_hbm.at[dst2.at[r]], kb, sk)
        pltpu.async_copy(vh_hbm.at[dst2.at[r]], vb, sv)

    def _wait_g(qb, kb, vb, sq, sk, sv):
        pltpu.make_async_copy(qh_hbm.at[src2.at[0]], qb, sq).wait()
        pltpu.make_async_copy(kh_hbm.at[dst2.at[0]], kb, sk).wait()
        pltpu.make_async_copy(vh_hbm.at[dst2.at[0]], vb, sv).wait()

    def _compute(r, qb, kb, vb, exbuf):
        ecol0 = r * C

        @plsc.parallel_loop(0, G * H, unroll=2)
        def _dots(i):
            g = i >> 3
            h = i & 7
            rows = g * 16 + iota
            hb = h * HD
            ecol = ecol0 + rows
            att0 = plsc.load_gather(bbrv, [hb + iota])
            att1 = jnp.zeros((16,), jnp.float32)
            for f in range(0, DE, 2):
                r0 = (f + iota) & (DE - 1)
                r1 = (f + 1 + iota) & (DE - 1)
                att0 = att0 + (plsc.load_gather(edges2, [r0, ecol])
                               * plsc.load_gather(wbtv, [hb + r0]))
                att1 = att1 + (plsc.load_gather(edges2, [r1, ecol])
                               * plsc.load_gather(wbtv, [hb + r1]))
            for d in range(0, HD, 2):
                c0 = ((d + iota) & (HD - 1)) + hb
                c1 = ((d + 1 + iota) & (HD - 1)) + hb
                att0 = att0 + (plsc.load_gather(qb, [rows, c0])
                               * plsc.load_gather(kb, [rows, c0]))
                att1 = att1 + (plsc.load_gather(qb, [rows, c1])
                               * plsc.load_gather(kb, [rows, c1]))
            ex = jnp.exp(att0 + att1)
            plsc.store_scatter(exbuf, [rows, jnp.full((16,), 0, jnp.int32) + h], ex)

        @plsc.parallel_loop(0, G * H, unroll=2)
        def _scale(i):
            g = i >> 3
            h = i & 7
            rows = g * 16 + iota
            hb = h * HD
            ex = plsc.load_gather(exbuf, [rows, jnp.full((16,), 0, jnp.int32) + h])
            for d in range(HD):
                col = ((d + iota) & (HD - 1)) + hb
                vc = plsc.load_gather(vb, [rows, col])
                plsc.store_scatter(vb, [rows, col], vc * ex)

    def _scat(r, vb, exbuf, sa, sd):
        pltpu.async_copy(vb, acc_sh.at[src2.at[r]], sa, add=True)
        pltpu.async_copy(exbuf, den_sh.at[src2.at[r]], sd, add=True)

    def _wait_scat(vb, exbuf, sa, sd):
        pltpu.make_async_copy(vb, acc_sh.at[src2.at[0]], sa).wait()
        pltpu.make_async_copy(exbuf, den_sh.at[src2.at[0]], sd).wait()

    def _sup(k, _):
        off = base_chunk + k * SUP
        pltpu.sync_copy(src2_hbm.at[pl.ds(off, SUP)], src2)
        pltpu.sync_copy(dst2_hbm.at[pl.ds(off, SUP)], dst2)
        pltpu.sync_copy(edgesT_hbm.at[:, pl.ds(off * C, SUP * C)], edges2)
        _issue(0, qA, kA, vA, gq0, gk0, gv0)
        _issue(1, qB, kB, vB, gq1, gk1, gv1)

        def _pair(j, _):
            a = 2 * j
            _wait_g(qA, kA, vA, gq0, gk0, gv0)
            _compute(a, qA, kA, vA, exA)
            _scat(a, vA, exA, sac0, sde0)
            _wait_g(qB, kB, vB, gq1, gk1, gv1)
            _wait_scat(vA, exA, sac0, sde0)

            @pl.when(j < NPAIR - 1)
            def _():
                _issue(a + 2, qA, kA, vA, gq0, gk0, gv0)

            _compute(a + 1, qB, kB, vB, exB)
            _scat(a + 1, vB, exB, sac1, sde1)
            _wait_scat(vB, exB, sac1, sde1)

            @pl.when(j < NPAIR - 1)
            def _():
                _issue(a + 3, qB, kB, vB, gq1, gk1, gv1)

            return 0

        lax.fori_loop(0, NPAIR, _pair, 0)
        return 0

    lax.fori_loop(0, NSUP, _sup, 0)

    @pl.when(tile < 16)
    def _():
        off = base_chunk + BCH
        pltpu.sync_copy(src2_hbm.at[pl.ds(off, 1)], src2.at[pl.ds(0, 1)])
        pltpu.sync_copy(dst2_hbm.at[pl.ds(off, 1)], dst2.at[pl.ds(0, 1)])
        pltpu.sync_copy(edgesT_hbm.at[:, pl.ds(off * C, C)],
                        edges2.at[:, pl.ds(0, C)])
        _issue(0, qA, kA, vA, gq0, gk0, gv0)
        _wait_g(qA, kA, vA, gq0, gk0, gv0)
        _compute(0, qA, kA, vA, exA)
        _scat(0, vA, exA, sac0, sde0)
        _wait_scat(vA, exA, sac0, sde0)

    plsc.subcore_barrier()

    pltpu.sync_copy(acc_sh.at[pl.ds(row0, RS)], acc_out.at[c, pl.ds(row0, RS)])
    pltpu.sync_copy(den_sh.at[pl.ds(row0, RS)], den_out.at[c, pl.ds(row0, RS)])


def _sc_pass(qh, kh, vh, edgesT, wbt, bbr, src, dst):
    mesh = plsc.VectorSubcoreMesh(core_axis_name="c", subcore_axis_name="s")
    f = pl.kernel(
        _sc_body,
        out_type=(jax.ShapeDtypeStruct((NC, NP_, DF), jnp.float32),
                  jax.ShapeDtypeStruct((NC, NP_, 2 * H), jnp.float32)),
        mesh=mesh,
        compiler_params=pltpu.CompilerParams(needs_layout_passes=False,
                                             use_tc_tiling_on_sc=False),
        scratch_types=[
            pltpu.VMEM((SUP, C), jnp.int32),
            pltpu.VMEM((SUP, C), jnp.int32),
            pltpu.VMEM((DE, SUP * C), jnp.float32),
            pltpu.VMEM((H * HD,), jnp.float32),
            pltpu.VMEM((H * HD,), jnp.float32),
            pltpu.VMEM((C, DF), jnp.float32),
            pltpu.VMEM((C, DF), jnp.float32),
            pltpu.VMEM((C, DF), jnp.float32),
            pltpu.VMEM((C, DF), jnp.float32),
            pltpu.VMEM((C, DF), jnp.float32),
            pltpu.VMEM((C, DF), jnp.float32),
            pltpu.VMEM((C, 2 * H), jnp.float32),
            pltpu.VMEM((C, 2 * H), jnp.float32),
            pltpu.VMEM_SHARED((NP_, DF), jnp.float32),
            pltpu.VMEM_SHARED((NP_, 2 * H), jnp.float32),
        ] + [pltpu.SemaphoreType.DMA] * 10,
    )
    return f(qh, kh, vh, edgesT, wbt, bbr,
             src.reshape(E // C, C), dst.reshape(E // C, C))


def _final_body(acc_ref, den_ref, r_ref, wo_ref, bo_ref, o_ref):
    a = acc_ref[0] + acc_ref[1]
    dn = den_ref[0] + den_ref[1]
    dr = jnp.dot(dn, r_ref[...], preferred_element_type=jnp.float32)
    dr = jnp.where(dr == 0.0, 1.0, dr)
    o = a / dr
    o_ref[...] = jnp.dot(o, wo_ref[...],
                         preferred_element_type=jnp.float32) + bo_ref[...]


def _finalize(acc, den, Wo, bo):
    BN = 2000
    grid = (N // BN,)
    rep = jnp.asarray(
        np.vstack([np.kron(np.eye(H), np.ones((1, HD))),
                   np.zeros((H, DF))]), dtype=jnp.float32)
    return pl.pallas_call(
        _final_body,
        grid=grid,
        in_specs=[pl.BlockSpec((NC, BN, DF), lambda i: (0, i, 0)),
                  pl.BlockSpec((NC, BN, 2 * H), lambda i: (0, i, 0)),
                  pl.BlockSpec((2 * H, DF), lambda i: (0, 0)),
                  pl.BlockSpec((DF, DF), lambda i: (0, 0)),
                  pl.BlockSpec((1, DF), lambda i: (0, 0))],
        out_specs=pl.BlockSpec((BN, DF), lambda i: (i, 0)),
        out_shape=jax.ShapeDtypeStruct((N, DF), jnp.float32),
    )(acc, den, rep, Wo, bo.reshape(1, DF))


def kernel(q, k, v, edges, edge_index, Wq, Wk, Wv, Wo, bo, Wb, bb):
    src = edge_index[:, 0]
    dst = edge_index[:, 1]
    qh, kh, vh = _proj(q, k, v, Wq, Wk, Wv)
    wbt = Wb.T.reshape(H * HD)
    bbr = jnp.repeat(bb, HD)
    acc, den = _sc_pass(qh, kh, vh, edges.T, wbt, bbr, src, dst)
    return _finalize(acc, den, Wo, bo)


# speedup vs baseline: 1.2113x; 1.0003x over previous
import functools

import jax
import jax.numpy as jnp
import numpy as np
from jax import lax
from jax.experimental import pallas as pl
from jax.experimental.pallas import tpu as pltpu
from jax.experimental.pallas import tpu_sc as plsc

N = 10000
E = 320000
DF = 128
DE = 16
H = 8
HD = 16
SCALE = HD ** (-0.5)

NC = 2
NS = 16
NT = NC * NS
C = 32
G = C // 16
SUP = 12
NPAIR = SUP // 2
NSUP = 26
BCH = NSUP * SUP
NP_ = 10112
RS = NP_ // NS


def _proj_body(q_ref, k_ref, v_ref, wq_ref, wk_ref, wv_ref,
               qh_ref, kh_ref, vh_ref):
    qh_ref[...] = jnp.dot(q_ref[...], wq_ref[...],
                          preferred_element_type=jnp.float32) * SCALE
    kh_ref[...] = jnp.dot(k_ref[...], wk_ref[...],
                          preferred_element_type=jnp.float32)
    vh_ref[...] = jnp.dot(v_ref[...], wv_ref[...],
                          preferred_element_type=jnp.float32)


def _proj(q, k, v, Wq, Wk, Wv):
    BN = 2000
    grid = (N // BN,)
    bspec_x = pl.BlockSpec((BN, DF), lambda i: (i, 0))
    bspec_w = pl.BlockSpec((DF, DF), lambda i: (0, 0))
    return pl.pallas_call(
        _proj_body,
        grid=grid,
        in_specs=[bspec_x, bspec_x, bspec_x, bspec_w, bspec_w, bspec_w],
        out_specs=[bspec_x, bspec_x, bspec_x],
        out_shape=[jax.ShapeDtypeStruct((N, DF), jnp.float32)] * 3,
    )(q, k, v, Wq, Wk, Wv)


def _sc_body(qh_hbm, kh_hbm, vh_hbm, edgesT_hbm, wbt_hbm, bbr_hbm,
             src2_hbm, dst2_hbm,
             acc_out, den_out,
             src2, dst2, edges2, wbtv, bbrv, qA, kA, vA, qB, kB, vB, exA, exB,
             acc_sh, den_sh,
             gq0, gk0, gv0, gq1, gk1, gv1, sac0, sde0, sac1, sde1):
    c = lax.axis_index("c")
    s = lax.axis_index("s")
    tile = c * NS + s
    base_chunk = tile * BCH + jnp.minimum(tile, 16)

    iota = lax.iota(jnp.int32, 16)
    zero16 = jnp.zeros((16,), jnp.float32)

    def _zero_vrow(r, _):
        for j in range(DF // 16):
            vA[r, pl.ds(j * 16, 16)] = zero16
        exA[r, pl.ds(0, 16)] = zero16
        exB[r, pl.ds(0, 16)] = zero16
        return 0

    lax.fori_loop(0, C, _zero_vrow, 0)
    pltpu.sync_copy(wbt_hbm, wbtv)
    pltpu.sync_copy(bbr_hbm, bbrv)

    row0 = s * RS
    for b in range(RS // C):
        pltpu.sync_copy(vA, acc_sh.at[pl.ds(row0 + b * C, C)])
        pltpu.sync_copy(exA, den_sh.at[pl.ds(row0 + b * C, C)])
    rtail = RS % C
    pltpu.sync_copy(vA.at[pl.ds(0, rtail)],
                    acc_sh.at[pl.ds(row0 + RS - rtail, rtail)])
    pltpu.sync_copy(exA.at[pl.ds(0, rtail)],
                    den_sh.at[pl.ds(row0 + RS - rtail, rtail)])
    plsc.subcore_barrier()

    def _issue(r, qb, kb, vb, sq, sk, sv):
        pltpu.async_copy(qh_hbm.at[src2.at[r]], qb, sq)
        pltpu.async_copy(kh_hbm.at[dst2.at[r]], kb, sk)
        pltpu.async_copy(vh_hbm.at[dst2.at[r]], vb, sv)

    def _wait_g(qb, kb, vb, sq, sk, sv):
        pltpu.make_async_copy(qh_hbm.at[src2.at[0]], qb, sq).wait()
        pltpu.make_async_copy(kh_hbm.at[dst2.at[0]], kb, sk).wait()
        pltpu.make_async_copy(vh_hbm.at[dst2.at[0]], vb, sv).wait()

    def _compute(r, qb, kb, vb, exbuf):
        ecol0 = r * C

        @plsc.parallel_loop(0, G * H, unroll=2)
        def _dots(i):
            g = i >> 3
            h = i & 7
            rows = g * 16 + iota
            hb = h * HD
            ecol = ecol0 + rows
            att0 = plsc.load_gather(bbrv, [hb + iota])
            att1 = jnp.zeros((16,), jnp.float32)
            for f in range(0, DE, 2):
                r0 = (f + iota) & (DE - 1)
                r1 = (f + 1 + iota) & (DE - 1)
                att0 = att0 + (plsc.load_gather(edges2, [r0, ecol])
                               * plsc.load_gather(wbtv, [hb + r0]))
                att1 = att1 + (plsc.load_gather(edges2, [r1, ecol])
                               * plsc.load_gather(wbtv, [hb + r1]))
            for d in range(0, HD, 2):
                c0 = ((d + iota) & (HD - 1)) + hb
                c1 = ((d + 1 + iota) & (HD - 1)) + hb
                att0 = att0 + (plsc.load_gather(qb, [rows, c0])
                               * plsc.load_gather(kb, [rows, c0]))
                att1 = att1 + (plsc.load_gather(qb, [rows, c1])
                               * plsc.load_gather(kb, [rows, c1]))
            ex = jnp.exp(att0 + att1)
            plsc.store_scatter(exbuf, [rows, jnp.full((16,), 0, jnp.int32) + h], ex)

        @plsc.parallel_loop(0, G * H, unroll=2)
        def _scale(i):
            g = i >> 3
            h = i & 7
            rows = g * 16 + iota
            hb = h * HD
            ex = plsc.load_gather(exbuf, [rows, jnp.full((16,), 0, jnp.int32) + h])
            for d in range(HD):
                col = ((d + iota) & (HD - 1)) + hb
                vc = plsc.load_gather(vb, [rows, col])
                plsc.store_scatter(vb, [rows, col], vc * ex)

    def _scat(r, vb, exbuf, sa, sd):
        pltpu.async_copy(vb, acc_sh.at[src2.at[r]], sa, add=True)
        pltpu.async_copy(exbuf, den_sh.at[src2.at[r]], sd, add=True)

    def _wait_scat(vb, exbuf, sa, sd):
        pltpu.make_async_copy(vb, acc_sh.at[src2.at[0]], sa).wait()
        pltpu.make_async_copy(exbuf, den_sh.at[src2.at[0]], sd).wait()

    def _sup(k, _):
        off = base_chunk + k * SUP
        pltpu.sync_copy(src2_hbm.at[pl.ds(off, SUP)], src2)
        pltpu.sync_copy(dst2_hbm.at[pl.ds(off, SUP)], dst2)
        pltpu.sync_copy(edgesT_hbm.at[:, pl.ds(off * C, SUP * C)], edges2)
        _issue(0, qA, kA, vA, gq0, gk0, gv0)
        _issue(1, qB, kB, vB, gq1, gk1, gv1)

        def _pair(j, _):
            a = 2 * j
            _wait_g(qA, kA, vA, gq0, gk0, gv0)
            _compute(a, qA, kA, vA, exA)
            _scat(a, vA, exA, sac0, sde0)
            _wait_g(qB, kB, vB, gq1, gk1, gv1)
            _wait_scat(vA, exA, sac0, sde0)

            @pl.when(j < NPAIR - 1)
            def _():
                _issue(a + 2, qA, kA, vA, gq0, gk0, gv0)

            _compute(a + 1, qB, kB, vB, exB)
            _scat(a + 1, vB, exB, sac1, sde1)
            _wait_scat(vB, exB, sac1, sde1)

            @pl.when(j < NPAIR - 1)
            def _():
                _issue(a + 3, qB, kB, vB, gq1, gk1, gv1)

            return 0

        lax.fori_loop(0, NPAIR, _pair, 0)
        return 0

    lax.fori_loop(0, NSUP, _sup, 0)

    @pl.when(tile < 16)
    def _():
        off = base_chunk + BCH
        pltpu.sync_copy(src2_hbm.at[pl.ds(off, 1)], src2.at[pl.ds(0, 1)])
        pltpu.sync_copy(dst2_hbm.at[pl.ds(off, 1)], dst2.at[pl.ds(0, 1)])
        pltpu.sync_copy(edgesT_hbm.at[:, pl.ds(off * C, C)],
                        edges2.at[:, pl.ds(0, C)])
        _issue(0, qA, kA, vA, gq0, gk0, gv0)
        _wait_g(qA, kA, vA, gq0, gk0, gv0)
        _compute(0, qA, kA, vA, exA)
        _scat(0, vA, exA, sac0, sde0)
        _wait_scat(vA, exA, sac0, sde0)

    plsc.subcore_barrier()

    pltpu.sync_copy(acc_sh.at[pl.ds(row0, RS)], acc_out.at[c, pl.ds(row0, RS)])
    pltpu.sync_copy(den_sh.at[pl.ds(row0, RS)], den_out.at[c, pl.ds(row0, RS)])


def _sc_pass(qh, kh, vh, edgesT, wbt, bbr, src, dst):
    mesh = plsc.VectorSubcoreMesh(core_axis_name="c", subcore_axis_name="s")
    f = pl.kernel(
        _sc_body,
        out_type=(jax.ShapeDtypeStruct((NC, NP_, DF), jnp.float32),
                  jax.ShapeDtypeStruct((NC, NP_, 2 * H), jnp.float32)),
        mesh=mesh,
        compiler_params=pltpu.CompilerParams(needs_layout_passes=False,
                                             use_tc_tiling_on_sc=False),
        scratch_types=[
            pltpu.VMEM((SUP, C), jnp.int32),
            pltpu.VMEM((SUP, C), jnp.int32),
            pltpu.VMEM((DE, SUP * C), jnp.float32),
            pltpu.VMEM((H * HD,), jnp.float32),
            pltpu.VMEM((H * HD,), jnp.float32),
            pltpu.VMEM((C, DF), jnp.float32),
            pltpu.VMEM((C, DF), jnp.float32),
            pltpu.VMEM((C, DF), jnp.float32),
            pltpu.VMEM((C, DF), jnp.float32),
            pltpu.VMEM((C, DF), jnp.float32),
            pltpu.VMEM((C, DF), jnp.float32),
            pltpu.VMEM((C, 2 * H), jnp.float32),
            pltpu.VMEM((C, 2 * H), jnp.float32),
            pltpu.VMEM_SHARED((NP_, DF), jnp.float32),
            pltpu.VMEM_SHARED((NP_, 2 * H), jnp.float32),
        ] + [pltpu.SemaphoreType.DMA] * 10,
    )
    return f(qh, kh, vh, edgesT, wbt, bbr,
             src.reshape(E // C, C), dst.reshape(E // C, C))


def _final_body(acc_ref, den_ref, r_ref, wo_ref, bo_ref, o_ref):
    a = acc_ref[0] + acc_ref[1]
    dn = den_ref[0] + den_ref[1]
    dr = jnp.dot(dn, r_ref[...], preferred_element_type=jnp.float32)
    dr = jnp.where(dr == 0.0, 1.0, dr)
    o = a / dr
    o_ref[...] = jnp.dot(o, wo_ref[...],
                         preferred_element_type=jnp.float32) + bo_ref[...]


def _finalize(acc, den, Wo, bo):
    BN = 2000
    grid = (N // BN,)
    rep = jnp.asarray(
        np.vstack([np.kron(np.eye(H), np.ones((1, HD))),
                   np.zeros((H, DF))]), dtype=jnp.float32)
    return pl.pallas_call(
        _final_body,
        grid=grid,
        in_specs=[pl.BlockSpec((NC, BN, DF), lambda i: (0, i, 0)),
                  pl.BlockSpec((NC, BN, 2 * H), lambda i: (0, i, 0)),
                  pl.BlockSpec((2 * H, DF), lambda i: (0, 0)),
                  pl.BlockSpec((DF, DF), lambda i: (0, 0)),
                  pl.BlockSpec((1, DF), lambda i: (0, 0))],
        out_specs=pl.BlockSpec((BN, DF), lambda i: (i, 0)),
        out_shape=jax.ShapeDtypeStruct((N, DF), jnp.float32),
    )(acc, den, rep, Wo, bo.reshape(1, DF))


def kernel(q, k, v, edges, edge_index, Wq, Wk, Wv, Wo, bo, Wb, bb):
    src = edge_index[:, 0]
    dst = edge_index[:, 1]
    qh, kh, vh = _proj(q, k, v, Wq, Wk, Wv)
    wbt = Wb.T.reshape(H * HD)
    bbr = jnp.repeat(bb, HD)
    acc, den = _sc_pass(qh, kh, vh, edges.T, wbt, bbr, src, dst)
    return _finalize(acc, den, Wo, bo)
